# Initial kernel scaffold; baseline (speedup 1.0000x reference)
import jax
import jax.numpy as jnp
from jax.experimental import pallas as pl


def kernel(x):
    raise NotImplementedError("write your pallas kernel here")


# speedup vs baseline: 6.8603x; 6.8603x over previous
import jax
import jax.numpy as jnp
from jax.experimental import pallas as pl
from jax.experimental.pallas import tpu as pltpu

_ROWS = 128
_COLS = 32768
_BC = 4096
_NC = _BC // 128


def _scan_kernel(x_ref, o_ref, carry_ref):
    c = pl.program_id(0)

    @pl.when(c == 0)
    def _():
        carry_ref[...] = jnp.zeros_like(carry_ref)

    x = x_ref[...]
    x3 = x.reshape(_ROWS, _NC, 128)

    k = jax.lax.broadcasted_iota(jnp.int32, (128, 128), 0)
    j = jax.lax.broadcasted_iota(jnp.int32, (128, 128), 1)
    tri = (k <= j).astype(jnp.float32)
    incl3 = jax.lax.dot_general(
        x3, tri, (((2,), (0,)), ((), ())),
        preferred_element_type=jnp.float32,
        precision=jax.lax.Precision.HIGHEST,
    )
    excl3 = incl3 - x3

    chunk_tot = incl3[:, :, 127]
    kk = jax.lax.broadcasted_iota(jnp.int32, (_NC, _NC), 0)
    jj = jax.lax.broadcasted_iota(jnp.int32, (_NC, _NC), 1)
    stri = (kk < jj).astype(jnp.float32)
    chunk_off = jax.lax.dot_general(
        chunk_tot, stri, (((1,), (0,)), ((), ())),
        preferred_element_type=jnp.float32,
        precision=jax.lax.Precision.HIGHEST,
    )

    carry = carry_ref[...]
    out3 = excl3 + chunk_off[:, :, None] + carry[:, :, None]
    o_ref[...] = out3.reshape(_ROWS, _BC)
    carry_ref[...] = carry + jnp.sum(chunk_tot, axis=1, keepdims=True)


def kernel(x):
    grid = (_COLS // _BC,)
    return pl.pallas_call(
        _scan_kernel,
        grid=grid,
        in_specs=[pl.BlockSpec((_ROWS, _BC), lambda c: (0, c))],
        out_specs=pl.BlockSpec((_ROWS, _BC), lambda c: (0, c)),
        out_shape=jax.ShapeDtypeStruct((_ROWS, _COLS), jnp.float32),
        scratch_shapes=[pltpu.VMEM((_ROWS, 1), jnp.float32)],
    )(x)


# speedup vs baseline: 11.0490x; 1.6106x over previous
import jax
import jax.numpy as jnp
from jax.experimental import pallas as pl
from jax.experimental.pallas import tpu as pltpu

_ROWS = 128
_COLS = 32768
_BC = 4096
_NC = _BC // 128


def _scan_kernel(x_ref, o_ref, carry_ref):
    c = pl.program_id(0)

    @pl.when(c == 0)
    def _():
        carry_ref[...] = jnp.zeros_like(carry_ref)

    x = x_ref[...]
    x3 = x.reshape(_ROWS, _NC, 128)

    k = jax.lax.broadcasted_iota(jnp.int32, (128, 128), 0)
    j = jax.lax.broadcasted_iota(jnp.int32, (128, 128), 1)
    tri = (k < j).astype(jnp.float32)
    excl3 = jax.lax.dot_general(
        x3, tri, (((2,), (0,)), ((), ())),
        preferred_element_type=jnp.float32,
    )

    chunk_tot = jnp.sum(x3, axis=2)
    kk = jax.lax.broadcasted_iota(jnp.int32, (_NC, _NC), 0)
    jj = jax.lax.broadcasted_iota(jnp.int32, (_NC, _NC), 1)
    stri = (kk < jj).astype(jnp.float32)
    chunk_off = jax.lax.dot_general(
        chunk_tot, stri, (((1,), (0,)), ((), ())),
        preferred_element_type=jnp.float32,
    )

    carry = carry_ref[...]
    out3 = excl3 + chunk_off[:, :, None] + carry[:, :, None]
    o_ref[...] = out3.reshape(_ROWS, _BC)
    carry_ref[...] = carry + jnp.sum(chunk_tot, axis=1, keepdims=True)


def kernel(x):
    grid = (_COLS // _BC,)
    return pl.pallas_call(
        _scan_kernel,
        grid=grid,
        in_specs=[pl.BlockSpec((_ROWS, _BC), lambda c: (0, c))],
        out_specs=pl.BlockSpec((_ROWS, _BC), lambda c: (0, c)),
        out_shape=jax.ShapeDtypeStruct((_ROWS, _COLS), jnp.float32),
        scratch_shapes=[pltpu.VMEM((_ROWS, 1), jnp.float32)],
    )(x)


# speedup vs baseline: 12.1727x; 1.1017x over previous
import jax
import jax.numpy as jnp
from jax.experimental import pallas as pl
from jax.experimental.pallas import tpu as pltpu

_ROWS = 128
_COLS = 32768
_BC = 8192
_NC = _BC // 128


def _scan_kernel(x_ref, o_ref, carry_ref):
    c = pl.program_id(0)

    @pl.when(c == 0)
    def _():
        carry_ref[...] = jnp.zeros_like(carry_ref)

    x = x_ref[...]
    x3 = x.reshape(_ROWS, _NC, 128)

    k = jax.lax.broadcasted_iota(jnp.int32, (128, 128), 0)
    j = jax.lax.broadcasted_iota(jnp.int32, (128, 128), 1)
    tri = (k < j).astype(jnp.float32)
    excl3 = jax.lax.dot_general(
        x3, tri, (((2,), (0,)), ((), ())),
        preferred_element_type=jnp.float32,
    )

    chunk_tot = jnp.sum(x3, axis=2)
    kk = jax.lax.broadcasted_iota(jnp.int32, (_NC, _NC), 0)
    jj = jax.lax.broadcasted_iota(jnp.int32, (_NC, _NC), 1)
    stri = (kk < jj).astype(jnp.float32)
    chunk_off = jax.lax.dot_general(
        chunk_tot, stri, (((1,), (0,)), ((), ())),
        preferred_element_type=jnp.float32,
    )

    carry = carry_ref[...]
    out3 = excl3 + chunk_off[:, :, None] + carry[:, :, None]
    o_ref[...] = out3.reshape(_ROWS, _BC)
    carry_ref[...] = carry + jnp.sum(chunk_tot, axis=1, keepdims=True)


def kernel(x):
    grid = (_COLS // _BC,)
    return pl.pallas_call(
        _scan_kernel,
        grid=grid,
        in_specs=[pl.BlockSpec((_ROWS, _BC), lambda c: (0, c))],
        out_specs=pl.BlockSpec((_ROWS, _BC), lambda c: (0, c)),
        out_shape=jax.ShapeDtypeStruct((_ROWS, _COLS), jnp.float32),
        scratch_shapes=[pltpu.VMEM((_ROWS, 1), jnp.float32)],
    )(x)
